# Initial kernel scaffold; baseline (speedup 1.0000x reference)
import jax
import jax.numpy as jnp
from jax.experimental import pallas as pl


def kernel(R, I, w1, b1, w2, b2, w3, b3, wu3, bu3, wu2, bu2, wu1, bu1, wf, bf, wfin, bfin):
    raise NotImplementedError("write your pallas kernel here")


# speedup vs baseline: 1.6646x; 1.6646x over previous
import functools

import jax
import jax.numpy as jnp
from jax import lax
from jax.experimental import pallas as pl
from jax.experimental.pallas import tpu as pltpu


def _conv_kernel(x_hbm, w_ref, b_ref, o_ref, xbuf, sem,
                 *, K, TH, Wo, Cin, relu):
    n = pl.program_id(0)
    t = pl.program_id(1)
    cp = pltpu.make_async_copy(x_hbm.at[n, pl.ds(t * TH, TH + K - 1)], xbuf, sem)
    cp.start()
    cp.wait()

    Cout = w_ref.shape[-1]
    acc = jnp.zeros((TH * Wo, Cout), jnp.float32)
    for kh in range(K):
        cols = [xbuf[kh:kh + TH, kw:kw + Wo, :] for kw in range(K)]
        patch = jnp.concatenate(cols, axis=-1).reshape(TH * Wo, K * Cin)
        acc = acc + jnp.dot(patch, w_ref[kh],
                            preferred_element_type=jnp.float32)
    acc = acc + b_ref[...].astype(jnp.float32)
    if relu:
        acc = jnp.maximum(acc, 0.0)
    o_ref[...] = acc.reshape(1, TH, Wo, Cout).astype(o_ref.dtype)


def _conv_valid(xp, w, b, *, relu, th):
    N, Hp, Wp, Cin = xp.shape
    K, _, _, Cout = w.shape
    Ho, Wo = Hp - K + 1, Wp - K + 1
    th = min(th, Ho)
    wg = w.reshape(K, K * Cin, Cout).astype(jnp.bfloat16)
    body = functools.partial(_conv_kernel, K=K, TH=th, Wo=Wo, Cin=Cin, relu=relu)
    return pl.pallas_call(
        body,
        out_shape=jax.ShapeDtypeStruct((N, Ho, Wo, Cout), jnp.bfloat16),
        grid=(N, Ho // th),
        in_specs=[
            pl.BlockSpec(memory_space=pl.ANY),
            pl.BlockSpec((K, K * Cin, Cout), lambda n, t: (0, 0, 0)),
            pl.BlockSpec((1, Cout), lambda n, t: (0, 0)),
        ],
        out_specs=pl.BlockSpec((1, th, Wo, Cout), lambda n, t: (n, t, 0, 0)),
        scratch_shapes=[
            pltpu.VMEM((th + K - 1, Wp, Cin), jnp.bfloat16),
            pltpu.SemaphoreType.DMA(()),
        ],
        compiler_params=pltpu.CompilerParams(
            dimension_semantics=("parallel", "arbitrary"),
            vmem_limit_bytes=48 * 1024 * 1024),
    )(xp, wg, b.reshape(1, Cout))


def _space_to_depth2(x):
    N, H, W, C = x.shape
    x = x.reshape(N, H // 2, 2, W // 2, 2, C)
    x = x.transpose(0, 1, 3, 2, 4, 5)
    return x.reshape(N, H // 2, W // 2, 4 * C)


def _depth_to_space2(x):
    N, H, W, C4 = x.shape
    C = C4 // 4
    x = x.reshape(N, H, W, 2, 2, C)
    x = x.transpose(0, 1, 3, 2, 4, 5)
    return x.reshape(N, 2 * H, 2 * W, C)


def _strided_conv4(x, w, b, *, relu, th):
    Cin, Cout = w.shape[2], w.shape[3]
    xp = jnp.pad(x, ((0, 0), (1, 1), (1, 1), (0, 0)))
    xs = _space_to_depth2(xp)
    wr = w.reshape(2, 2, 2, 2, Cin, Cout).transpose(0, 2, 1, 3, 4, 5)
    wr = wr.reshape(2, 2, 4 * Cin, Cout)
    return _conv_valid(xs, wr, b, relu=relu, th=th)


_PHASE_TAPS = {(0, 0): 3, (0, 1): 1, (1, 1): 2, (1, 2): 0}


def _tconv_weights3(w):
    Cin, Cout = w.shape[2], w.shape[3]
    w3 = jnp.zeros((3, 3, Cin, 2, 2, Cout), w.dtype)
    for (ry, a), kh in _PHASE_TAPS.items():
        for (rx, c), kw in _PHASE_TAPS.items():
            w3 = w3.at[a, c, :, ry, rx, :].set(w[kh, kw])
    return w3.reshape(3, 3, Cin, 4 * Cout)


def _tconv4(xp, w, b, *, th):
    y = _conv_valid(xp, _tconv_weights3(w), jnp.tile(b, 4), relu=True, th=th)
    return _depth_to_space2(y)


def _pad1(x):
    return jnp.pad(x, ((0, 0), (1, 1), (1, 1), (0, 0)))


def _tail_kernel(h1cp_hbm, p2r_hbm, p3r_hbm,
                 wc_ref, wt_ref, wx_ref, bf_ref, sc_ref, o_ref,
                 a1, a2, a3, s1, s2, s3, f00, f01, f10, f11,
                 *, TH, W):
    n = pl.program_id(0)
    t = pl.program_id(1)

    c1 = pltpu.make_async_copy(
        h1cp_hbm.at[n, pl.ds(t * (TH // 2), TH // 2 + 2)], a1, s1)
    c2 = pltpu.make_async_copy(
        p2r_hbm.at[n, pl.ds(t * (TH // 4), TH // 4 + 2)], a2, s2)
    c3 = pltpu.make_async_copy(
        p3r_hbm.at[n, pl.ds(t * (TH // 8), TH // 8 + 2)], a3, s3)
    c1.start(); c2.start(); c3.start()
    c1.wait(); c2.wait(); c3.wait()

    R1 = TH // 2 + 2
    W1 = a1.shape[1]
    W2 = W // 2
    M = TH // 2 + 1
    TH2 = TH // 2
    bu1 = sc_ref[0, 0]
    bfin = sc_ref[0, 1]

    z1 = jnp.dot(a1[...].reshape(R1 * W1, 128), wc_ref[...],
                 preferred_element_type=jnp.float32).reshape(R1, W1, 80)
    p1 = z1[:, 1:W1 - 1, :64]
    s_taps = z1[:, :, 64:80]

    def xr_phase(ry, rx):
        acc = None
        for dy in range(2):
            for dx in range(2):
                tap = 4 * (3 - ry - 2 * dy) + (3 - rx - 2 * dx)
                c = rx + dx
                v = s_taps[dy:dy + M, c:c + W2, tap:tap + 1]
                acc = v if acc is None else acc + v
        return jnp.maximum(acc + bu1, 0.0)

    def rep_lead(p, r, nm, head):
        mid = jnp.broadcast_to(p[1:1 + nm, None], (nm, r) + p.shape[1:])
        mid = mid.reshape((nm * r,) + p.shape[1:])
        if head:
            return jnp.concatenate([p[0:1], mid], axis=0)
        return jnp.concatenate([mid, p[nm + 1:nm + 2]], axis=0)

    wxv = wx_ref[...].reshape(1, 1, 64)
    bfv = bf_ref[...].reshape(1, 1, 64)
    row_ids = lax.broadcasted_iota(jnp.int32, (M, 1, 1), 0)
    fpl = {(1, 0): f10, (1, 1): f11, (0, 0): f00, (0, 1): f01}
    for ry in (1, 0):
        if ry == 1:
            q1 = p1[0:M]
            q2 = rep_lead(a2[...].astype(jnp.float32), 2, TH // 4, True)
            q3 = rep_lead(a3[...].astype(jnp.float32), 4, TH // 8, True)
            valid = (TH * t + 2 * row_ids) > 0
        else:
            q1 = p1[1:M + 1]
            q2 = rep_lead(a2[...].astype(jnp.float32), 2, TH // 4, False)
            q3 = rep_lead(a3[...].astype(jnp.float32), 4, TH // 8, False)
            valid = (TH * t + 2 * row_ids) < W
        base = q1 + q2 + q3 + bfv
        for rx in (0, 1):
            fph = xr_phase(ry, rx) * wxv + base
            fph = jnp.where(valid, fph, 0.0)
            buf = fpl[(ry, rx)]
            buf[:, 1:W2 + 1, :] = fph.astype(jnp.bfloat16)
            buf[:, 0:1, :] = jnp.zeros((M, 1, 64), jnp.bfloat16)
            buf[:, W2 + 1:W2 + 2, :] = jnp.zeros((M, 1, 64), jnp.bfloat16)

    tt = {}
    for ry in (0, 1):
        for rx in (0, 1):
            v = jnp.dot(fpl[(ry, rx)][...].reshape(M * (W2 + 2), 64),
                        wt_ref[...],
                        preferred_element_type=jnp.float32
                        ).reshape(M, W2 + 2, 16)
            tt[(ry, rx)] = jnp.transpose(v, (2, 0, 1))

    for qr in (0, 1):
        for qc in (0, 1):
            out = jnp.full((TH2, W2), bfin, jnp.float32)
            for kh in range(3):
                ryp = (qr + 1 + kh) % 2
                m0 = (qr + kh) // 2 if ryp == 0 else (qr + 1 + kh) // 2
                for kw in range(3):
                    rxp = (qc + 1 + kw) % 2
                    jj0 = ((qc + kw + 1) // 2 if rxp == 0
                           else (qc + kw) // 2)
                    out = out + tt[(ryp, rxp)][
                        3 * kh + kw, m0:m0 + TH2, jj0:jj0 + W2]
            o_ref[0, 2 * qr + qc] = out


def _tail(h1cp, p2r, p3r, wc, wt, wx, bf, scalars, *, TH=16):
    N = h1cp.shape[0]
    W = h1cp.shape[2] * 2 - 4
    nt = W // TH
    return pl.pallas_call(
        functools.partial(_tail_kernel, TH=TH, W=W),
        out_shape=jax.ShapeDtypeStruct((N, 4, W // 2, W // 2), jnp.float32),
        grid=(N, nt),
        in_specs=[
            pl.BlockSpec(memory_space=pl.ANY),
            pl.BlockSpec(memory_space=pl.ANY),
            pl.BlockSpec(memory_space=pl.ANY),
            pl.BlockSpec((128, 80), lambda n, t: (0, 0)),
            pl.BlockSpec((64, 16), lambda n, t: (0, 0)),
            pl.BlockSpec((1, 64), lambda n, t: (0, 0)),
            pl.BlockSpec((1, 64), lambda n, t: (0, 0)),
            pl.BlockSpec((1, 2), lambda n, t: (0, 0)),
        ],
        out_specs=pl.BlockSpec((1, 4, TH // 2, W // 2),
                               lambda n, t: (n, 0, t, 0)),
        scratch_shapes=[
            pltpu.VMEM((TH // 2 + 2, W // 2 + 2, 128), jnp.bfloat16),
            pltpu.VMEM((TH // 4 + 2, W // 2, 64), jnp.bfloat16),
            pltpu.VMEM((TH // 8 + 2, W // 2, 64), jnp.bfloat16),
            pltpu.SemaphoreType.DMA(()),
            pltpu.SemaphoreType.DMA(()),
            pltpu.SemaphoreType.DMA(()),
            pltpu.VMEM((TH // 2 + 1, W // 2 + 2, 64), jnp.bfloat16),
            pltpu.VMEM((TH // 2 + 1, W // 2 + 2, 64), jnp.bfloat16),
            pltpu.VMEM((TH // 2 + 1, W // 2 + 2, 64), jnp.bfloat16),
            pltpu.VMEM((TH // 2 + 1, W // 2 + 2, 64), jnp.bfloat16),
        ],
        compiler_params=pltpu.CompilerParams(
            dimension_semantics=("parallel", "arbitrary"),
            vmem_limit_bytes=40 * 1024 * 1024),
    )(h1cp, p2r, p3r, wc, wt, wx, bf, scalars)


def kernel(R, I, w1, b1, w2, b2, w3, b3, wu3, bu3, wu2, bu2,
           wu1, bu1, wf, bf, wfin, bfin):
    x = jnp.concatenate([R, I], axis=1)
    x = jnp.transpose(x, (0, 2, 3, 1)).astype(jnp.bfloat16)

    h1 = _strided_conv4(x, w1, b1, relu=True, th=64)
    h2 = _strided_conv4(h1, w2, b2, relu=True, th=32)
    h3 = _strided_conv4(h2, w3, b3, relu=False, th=64)

    h3p = _pad1(h3)
    u3 = _tconv4(h3p, wu3, bu3, th=64)
    h2c = jnp.concatenate([h2, u3], axis=-1)
    u2 = _tconv4(_pad1(h2c), wu2, bu2, th=32)
    h1cp = _pad1(jnp.concatenate([h1, u2], axis=-1))

    wx, wf1, wf2, wf3 = wf[0], wf[1:129], wf[129:257], wf[257:321]
    z64 = jnp.zeros((64,), jnp.float32)
    p2 = _conv_valid(h2c, wf2.reshape(1, 1, 128, 64), z64, relu=False, th=32)
    p3 = _conv_valid(h3, wf3.reshape(1, 1, 64, 64), z64, relu=False, th=64)
    p2r = jnp.pad(jnp.repeat(p2, 2, axis=2), ((0, 0), (1, 1), (0, 0), (0, 0)))
    p3r = jnp.pad(jnp.repeat(p3, 4, axis=2), ((0, 0), (1, 1), (0, 0), (0, 0)))

    wu = wu1[:, :, :, 0].transpose(2, 0, 1).reshape(128, 16)
    wc = jnp.concatenate([wf1, wu], axis=1).astype(jnp.bfloat16)
    wt = jnp.concatenate(
        [wfin[:, :, :, 0].transpose(2, 0, 1).reshape(64, 9),
         jnp.zeros((64, 7), jnp.float32)], axis=1).astype(jnp.bfloat16)
    scalars = jnp.stack([bu1[0], bfin[0]]).reshape(1, 2)

    o = _tail(h1cp, p2r, p3r, wc, wt, wx.reshape(1, 64), bf.reshape(1, 64),
              scalars)
    N, _, W2, _ = o.shape
    W = 2 * W2
    o = o.reshape(N, 2, 2, W2, W2).transpose(0, 3, 1, 4, 2).reshape(N, W, W)
    return o[:, None, :, :]
